# Initial kernel scaffold; baseline (speedup 1.0000x reference)
import jax
import jax.numpy as jnp
from jax.experimental import pallas as pl


def kernel(x, W):
    raise NotImplementedError("write your pallas kernel here")


# speedup vs baseline: 1.2945x; 1.2945x over previous
import functools

import jax
import jax.numpy as jnp
from jax import lax
from jax.experimental import pallas as pl
from jax.experimental.pallas import tpu as pltpu
from jax.experimental.pallas import tpu_sc as plsc

EMB_D = 300
EMB_D_PAD = 304
SEQ_L = 20
N_SEQ = 26624
N_WORKERS = 32
SEQ_PER_W = N_SEQ // N_WORKERS
G = 8
GT = G * SEQ_L
NG = SEQ_PER_W // G
N_CHUNK = 19

_CHUNK_OFF = tuple(min(16 * j, EMB_D - 16) for j in range(N_CHUNK))


def _emb_body(x_hbm, w_hbm, out_hbm, idx0, rows0, out_v):
    wid = lax.axis_index("c") * 16 + lax.axis_index("s")
    seq_base = wid * SEQ_PER_W
    lane = lax.broadcasted_iota(jnp.int32, (16,), 0)

    def load_idx(idx_v, g):
        tok0 = (seq_base + g * G) * SEQ_L
        pltpu.sync_copy(x_hbm.at[pl.ds(tok0, GT)], idx_v)

    def do_gather(idx_v, rows_v):
        pltpu.sync_copy(w_hbm.at[idx_v.at[pl.ds(0, 80)]],
                        rows_v.at[pl.ds(0, 80)])
        pltpu.sync_copy(w_hbm.at[idx_v.at[pl.ds(80, 80)]],
                        rows_v.at[pl.ds(80, 80)])

    def seq_body(s, carry, idx_v, rows_v):
        t0 = s * SEQ_L
        c1 = idx_v[pl.ds(t0, 16)]
        c2 = idx_v[pl.ds(t0 + 4, 16)]
        one = jnp.full((16,), 1.0, jnp.float32)
        zero = jnp.full((16,), 0.0, jnp.float32)
        m1 = jnp.where(c1 != 0, one, zero)
        m2 = jnp.where((c2 != 0) & (lane >= 12), one, zero)
        cnt = m1 + m2
        for sh in (1, 2, 4, 8):
            cnt = cnt + cnt.at[lane ^ sh].get(mode="promise_in_bounds")
        inv = one / jnp.maximum(cnt, one)
        accs = [jnp.zeros((16,), jnp.float32)] * N_CHUNK
        for l in range(SEQ_L):
            r = t0 + l
            for j in range(N_CHUNK):
                accs[j] = accs[j] + rows_v[r, pl.ds(_CHUNK_OFF[j], 16)]
        for j in range(N_CHUNK):
            out_v[s, pl.ds(_CHUNK_OFF[j], 16)] = accs[j] * inv
        return carry

    def group_body(g, carry):
        load_idx(idx0, g)
        do_gather(idx0, rows0)
        lax.fori_loop(0, G,
                      functools.partial(seq_body, idx_v=idx0, rows_v=rows0),
                      0)
        pltpu.sync_copy(out_v, out_hbm.at[pl.ds(seq_base + g * G, G)])
        return carry

    lax.fori_loop(0, NG, group_body, 0)


_emb = functools.partial(
    pl.kernel,
    out_type=jax.ShapeDtypeStruct((N_SEQ, EMB_D), jnp.float32),
    mesh=plsc.VectorSubcoreMesh(core_axis_name="c", subcore_axis_name="s"),
    scratch_types=[
        pltpu.VMEM((GT,), jnp.int32),
        pltpu.VMEM((GT, EMB_D_PAD), jnp.float32),
        pltpu.VMEM((G, EMB_D), jnp.float32),
    ],
    compiler_params=pltpu.CompilerParams(use_tc_tiling_on_sc=False),
)(_emb_body)


def kernel(x, W):
    b, nk, _ = x.shape
    w_pad = jnp.pad(W, ((0, 0), (0, EMB_D_PAD - EMB_D)))
    pooled = _emb(x.reshape(-1), w_pad)
    return pooled.reshape(b, nk, EMB_D)


# speedup vs baseline: 1.5813x; 1.2215x over previous
import functools

import jax
import jax.numpy as jnp
from jax import lax
from jax.experimental import pallas as pl
from jax.experimental.pallas import tpu as pltpu
from jax.experimental.pallas import tpu_sc as plsc

EMB_D = 300
EMB_D_PAD = 304
SEQ_L = 20
N_SEQ = 26624
N_WORKERS = 32
SEQ_PER_W = N_SEQ // N_WORKERS
G = 8
GT = G * SEQ_L
NG = SEQ_PER_W // G
N_CHUNK = 19

_CHUNK_OFF = tuple(min(16 * j, EMB_D - 16) for j in range(N_CHUNK))


def _emb_body(x_hbm, w_hbm, out_hbm, idx0, idx1, rows0, rows1, out_v,
              sem0, sem1):
    wid = lax.axis_index("c") * 16 + lax.axis_index("s")
    seq_base = wid * SEQ_PER_W
    lane = lax.broadcasted_iota(jnp.int32, (16,), 0)

    def load_idx(idx_v, g):
        tok0 = (seq_base + g * G) * SEQ_L
        pltpu.sync_copy(x_hbm.at[pl.ds(tok0, GT)], idx_v)

    def gather_descs(idx_v, rows_v, sem):
        return (pltpu.make_async_copy(w_hbm.at[idx_v.at[pl.ds(0, 80)]],
                                      rows_v.at[pl.ds(0, 80)], sem),
                pltpu.make_async_copy(w_hbm.at[idx_v.at[pl.ds(80, 80)]],
                                      rows_v.at[pl.ds(80, 80)], sem))

    def start_gather(idx_v, rows_v, sem):
        for d in gather_descs(idx_v, rows_v, sem):
            d.start()

    def wait_gather(idx_v, rows_v, sem):
        for d in gather_descs(idx_v, rows_v, sem):
            d.wait()

    def seq_body(s, carry, idx_v, rows_v):
        t0 = s * SEQ_L
        c1 = idx_v[pl.ds(t0, 16)]
        c2 = idx_v[pl.ds(t0 + 4, 16)]
        one = jnp.full((16,), 1.0, jnp.float32)
        zero = jnp.full((16,), 0.0, jnp.float32)
        m1 = jnp.where(c1 != 0, one, zero)
        m2 = jnp.where((c2 != 0) & (lane >= 12), one, zero)
        cnt = m1 + m2
        for sh in (1, 2, 4, 8):
            cnt = cnt + cnt.at[lane ^ sh].get(mode="promise_in_bounds")
        inv = one / jnp.maximum(cnt, one)
        accs = [jnp.zeros((16,), jnp.float32)] * N_CHUNK
        for l in range(SEQ_L):
            r = t0 + l
            for j in range(N_CHUNK):
                accs[j] = accs[j] + rows_v[r, pl.ds(_CHUNK_OFF[j], 16)]
        for j in range(N_CHUNK):
            out_v[s, pl.ds(_CHUNK_OFF[j], 16)] = accs[j] * inv
        return carry

    load_idx(idx0, 0)
    start_gather(idx0, rows0, sem0)
    load_idx(idx1, 1)
    start_gather(idx1, rows1, sem1)

    def pair_body(i, carry):
        for p in (0, 1):
            g = 2 * i + p
            idx_v = idx0 if p == 0 else idx1
            rows_v = rows0 if p == 0 else rows1
            sem = sem0 if p == 0 else sem1
            wait_gather(idx_v, rows_v, sem)
            lax.fori_loop(0, G,
                          functools.partial(seq_body, idx_v=idx_v,
                                            rows_v=rows_v), 0)
            pltpu.sync_copy(out_v, out_hbm.at[pl.ds(seq_base + g * G, G)])
            nxt = g + 2

            @pl.when(nxt < NG)
            def _():
                load_idx(idx_v, nxt)
                start_gather(idx_v, rows_v, sem)
        return carry

    lax.fori_loop(0, NG // 2, pair_body, 0)


_emb = functools.partial(
    pl.kernel,
    out_type=jax.ShapeDtypeStruct((N_SEQ, EMB_D), jnp.float32),
    mesh=plsc.VectorSubcoreMesh(core_axis_name="c", subcore_axis_name="s"),
    scratch_types=[
        pltpu.VMEM((GT,), jnp.int32),
        pltpu.VMEM((GT,), jnp.int32),
        pltpu.VMEM((GT, EMB_D_PAD), jnp.float32),
        pltpu.VMEM((GT, EMB_D_PAD), jnp.float32),
        pltpu.VMEM((G, EMB_D), jnp.float32),
        pltpu.SemaphoreType.DMA,
        pltpu.SemaphoreType.DMA,
    ],
    compiler_params=pltpu.CompilerParams(use_tc_tiling_on_sc=False),
)(_emb_body)


def kernel(x, W):
    b, nk, _ = x.shape
    w_pad = jnp.pad(W, ((0, 0), (0, EMB_D_PAD - EMB_D)))
    pooled = _emb(x.reshape(-1), w_pad)
    return pooled.reshape(b, nk, EMB_D)


# speedup vs baseline: 1.6511x; 1.0442x over previous
import functools

import jax
import jax.numpy as jnp
from jax import lax
from jax.experimental import pallas as pl
from jax.experimental.pallas import tpu as pltpu
from jax.experimental.pallas import tpu_sc as plsc

EMB_D = 300
EMB_D_PAD = 304
SEQ_L = 20
N_SEQ = 26624
N_WORKERS = 32
SEQ_PER_W = N_SEQ // N_WORKERS
G = 8
GT = G * SEQ_L
NG = SEQ_PER_W // G
N_CHUNK = 19

_CHUNK_OFF = tuple(min(16 * j, EMB_D - 16) for j in range(N_CHUNK))


def _emb_body(x_hbm, w_hbm, out_hbm, idx_all, rows0, rows1, out0, out1,
              sem0, sem1, osem0, osem1):
    wid = lax.axis_index("c") * 16 + lax.axis_index("s")
    seq_base = wid * SEQ_PER_W
    lane = lax.broadcasted_iota(jnp.int32, (16,), 0)

    def gather_descs(g, rows_v, sem):
        return (
            pltpu.make_async_copy(
                w_hbm.at[idx_all.at[pl.ds(g * GT, 80)]],
                rows_v.at[pl.ds(0, 80)], sem),
            pltpu.make_async_copy(
                w_hbm.at[idx_all.at[pl.ds(g * GT + 80, 80)]],
                rows_v.at[pl.ds(80, 80)], sem),
        )

    def start_gather(g, rows_v, sem):
        for d in gather_descs(g, rows_v, sem):
            d.start()

    def wait_gather(g, rows_v, sem):
        for d in gather_descs(g, rows_v, sem):
            d.wait()

    def out_desc(g, out_v, osem):
        return pltpu.make_async_copy(
            out_v, out_hbm.at[pl.ds(seq_base + g * G, G)], osem)

    pltpu.sync_copy(x_hbm.at[pl.ds(seq_base * SEQ_L, SEQ_PER_W * SEQ_L)],
                    idx_all)
    start_gather(0, rows0, sem0)
    start_gather(1, rows1, sem1)

    def seq_body(s, carry, g, rows_v, out_v):
        t0 = g * GT + s * SEQ_L
        c1 = idx_all[pl.ds(t0, 16)]
        c2 = idx_all[pl.ds(t0 + 4, 16)]
        one = jnp.full((16,), 1.0, jnp.float32)
        zero = jnp.full((16,), 0.0, jnp.float32)
        m1 = jnp.where(c1 != 0, one, zero)
        m2 = jnp.where((c2 != 0) & (lane >= 12), one, zero)
        cnt = m1 + m2
        for sh in (1, 2, 4, 8):
            cnt = cnt + cnt.at[lane ^ sh].get(mode="promise_in_bounds")
        inv = one / jnp.maximum(cnt, one)
        r0 = s * SEQ_L
        accs = [jnp.zeros((16,), jnp.float32)] * N_CHUNK
        for l in range(SEQ_L):
            r = r0 + l
            for j in range(N_CHUNK):
                accs[j] = accs[j] + rows_v[r, pl.ds(_CHUNK_OFF[j], 16)]
        for j in range(N_CHUNK):
            out_v[s, pl.ds(_CHUNK_OFF[j], 16)] = accs[j] * inv
        return carry

    def pair_body(i, carry):
        for p in (0, 1):
            g = 2 * i + p
            rows_v = rows0 if p == 0 else rows1
            sem = sem0 if p == 0 else sem1
            out_v = out0 if p == 0 else out1
            osem = osem0 if p == 0 else osem1
            wait_gather(g, rows_v, sem)

            @pl.when(g >= 2)
            def _():
                out_desc(g - 2, out_v, osem).wait()

            lax.fori_loop(0, G,
                          functools.partial(seq_body, g=g, rows_v=rows_v,
                                            out_v=out_v), 0)
            out_desc(g, out_v, osem).start()
            nxt = g + 2

            @pl.when(nxt < NG)
            def _():
                start_gather(nxt, rows_v, sem)
        return carry

    lax.fori_loop(0, NG // 2, pair_body, 0)
    out_desc(NG - 2, out0, osem0).wait()
    out_desc(NG - 1, out1, osem1).wait()


_emb = functools.partial(
    pl.kernel,
    out_type=jax.ShapeDtypeStruct((N_SEQ, EMB_D), jnp.float32),
    mesh=plsc.VectorSubcoreMesh(core_axis_name="c", subcore_axis_name="s"),
    scratch_types=[
        pltpu.VMEM((SEQ_PER_W * SEQ_L,), jnp.int32),
        pltpu.VMEM((GT, EMB_D_PAD), jnp.float32),
        pltpu.VMEM((GT, EMB_D_PAD), jnp.float32),
        pltpu.VMEM((G, EMB_D), jnp.float32),
        pltpu.VMEM((G, EMB_D), jnp.float32),
        pltpu.SemaphoreType.DMA,
        pltpu.SemaphoreType.DMA,
        pltpu.SemaphoreType.DMA,
        pltpu.SemaphoreType.DMA,
    ],
    compiler_params=pltpu.CompilerParams(use_tc_tiling_on_sc=False),
)(_emb_body)


def kernel(x, W):
    b, nk, _ = x.shape
    w_pad = jnp.pad(W, ((0, 0), (0, EMB_D_PAD - EMB_D)))
    pooled = _emb(x.reshape(-1), w_pad)
    return pooled.reshape(b, nk, EMB_D)


# speedup vs baseline: 1.6737x; 1.0137x over previous
import functools

import jax
import jax.numpy as jnp
from jax import lax
from jax.experimental import pallas as pl
from jax.experimental.pallas import tpu as pltpu
from jax.experimental.pallas import tpu_sc as plsc

EMB_D = 300
EMB_D_PAD = 304
SEQ_L = 20
N_SEQ = 26624
N_WORKERS = 32
SEQ_PER_W = N_SEQ // N_WORKERS
G = 8
GT = G * SEQ_L
NG = SEQ_PER_W // G
N_CHUNK = 19

_CHUNK_OFF = tuple(min(16 * j, EMB_D - 16) for j in range(N_CHUNK))


def _emb_body(x_hbm, w_hbm, out_hbm, idx_all, rows0, rows1, out0, out1,
              sem0, sem1, osem0, osem1):
    wid = lax.axis_index("c") * 16 + lax.axis_index("s")
    seq_base = wid * SEQ_PER_W
    lane = lax.broadcasted_iota(jnp.int32, (16,), 0)

    def gather_descs(g, rows_v, sem):
        return (
            pltpu.make_async_copy(
                w_hbm.at[idx_all.at[pl.ds(g * GT, 80)]],
                rows_v.at[pl.ds(0, 80)], sem),
            pltpu.make_async_copy(
                w_hbm.at[idx_all.at[pl.ds(g * GT + 80, 80)]],
                rows_v.at[pl.ds(80, 80)], sem),
        )

    def start_gather(g, rows_v, sem):
        for d in gather_descs(g, rows_v, sem):
            d.start()

    def wait_gather(g, rows_v, sem):
        for d in gather_descs(g, rows_v, sem):
            d.wait()

    def out_desc(g, out_v, osem):
        return pltpu.make_async_copy(
            out_v, out_hbm.at[pl.ds((seq_base + g * G) * EMB_D, G * EMB_D)],
            osem)

    pltpu.sync_copy(x_hbm.at[pl.ds(seq_base * SEQ_L, SEQ_PER_W * SEQ_L)],
                    idx_all)
    start_gather(0, rows0, sem0)
    start_gather(1, rows1, sem1)

    def seq_body(s, carry, g, rows_v, out_v):
        t0 = g * GT + s * SEQ_L
        c1 = idx_all[pl.ds(t0, 16)]
        c2 = idx_all[pl.ds(t0 + 4, 16)]
        one = jnp.full((16,), 1.0, jnp.float32)
        zero = jnp.full((16,), 0.0, jnp.float32)
        m1 = jnp.where(c1 != 0, one, zero)
        m2 = jnp.where((c2 != 0) & (lane >= 12), one, zero)
        cnt = m1 + m2
        for sh in (1, 2, 4, 8):
            cnt = cnt + cnt.at[lane ^ sh].get(mode="promise_in_bounds")
        inv = one / jnp.maximum(cnt, one)
        r0 = s * SEQ_L
        accs = [jnp.zeros((16,), jnp.float32)] * N_CHUNK
        for l in range(SEQ_L):
            r = r0 + l
            for j in range(N_CHUNK):
                accs[j] = accs[j] + rows_v[r, pl.ds(_CHUNK_OFF[j], 16)]
        o0 = s * EMB_D
        for j in range(N_CHUNK):
            out_v[pl.ds(o0 + _CHUNK_OFF[j], 16)] = accs[j] * inv
        return carry

    def pair_body(i, carry):
        for p in (0, 1):
            g = 2 * i + p
            rows_v = rows0 if p == 0 else rows1
            sem = sem0 if p == 0 else sem1
            out_v = out0 if p == 0 else out1
            osem = osem0 if p == 0 else osem1
            wait_gather(g, rows_v, sem)

            @pl.when(g >= 2)
            def _():
                out_desc(g - 2, out_v, osem).wait()

            lax.fori_loop(0, G,
                          functools.partial(seq_body, g=g, rows_v=rows_v,
                                            out_v=out_v), 0)
            out_desc(g, out_v, osem).start()
            nxt = g + 2

            @pl.when(nxt < NG)
            def _():
                start_gather(nxt, rows_v, sem)
        return carry

    lax.fori_loop(0, NG // 2, pair_body, 0)
    out_desc(NG - 2, out0, osem0).wait()
    out_desc(NG - 1, out1, osem1).wait()


_emb = functools.partial(
    pl.kernel,
    out_type=jax.ShapeDtypeStruct((N_SEQ * EMB_D,), jnp.float32),
    mesh=plsc.VectorSubcoreMesh(core_axis_name="c", subcore_axis_name="s"),
    scratch_types=[
        pltpu.VMEM((SEQ_PER_W * SEQ_L,), jnp.int32),
        pltpu.VMEM((GT, EMB_D_PAD), jnp.float32),
        pltpu.VMEM((GT, EMB_D_PAD), jnp.float32),
        pltpu.VMEM((G * EMB_D,), jnp.float32),
        pltpu.VMEM((G * EMB_D,), jnp.float32),
        pltpu.SemaphoreType.DMA,
        pltpu.SemaphoreType.DMA,
        pltpu.SemaphoreType.DMA,
        pltpu.SemaphoreType.DMA,
    ],
    compiler_params=pltpu.CompilerParams(use_tc_tiling_on_sc=False),
)(_emb_body)


def kernel(x, W):
    b, nk, _ = x.shape
    w_pad = jnp.pad(W, ((0, 0), (0, EMB_D_PAD - EMB_D)))
    pooled = _emb(x.reshape(-1), w_pad)
    return pooled.reshape(b, nk, EMB_D)


# speedup vs baseline: 1.9973x; 1.1933x over previous
import functools

import jax
import jax.numpy as jnp
from jax import lax
from jax.experimental import pallas as pl
from jax.experimental.pallas import tpu as pltpu
from jax.experimental.pallas import tpu_sc as plsc

EMB_D = 300
EMB_D_PAD = 304
SEQ_L = 20
N_SEQ = 26624
N_WORKERS = 32
SEQ_PER_W = N_SEQ // N_WORKERS
G = 8
GT = G * SEQ_L
NG = SEQ_PER_W // G
N_CHUNK = 19

_CHUNK_OFF = tuple(min(16 * j, EMB_D - 16) for j in range(N_CHUNK))


def _emb_body(x_hbm, w_hbm, out_hbm, idx_all, rows0, rows1, out0, out1,
              sem0, sem1, osem0, osem1):
    wid = lax.axis_index("c") * 16 + lax.axis_index("s")
    seq_base = wid * SEQ_PER_W
    lane = lax.broadcasted_iota(jnp.int32, (16,), 0)

    def gather_descs(g, rows_v, sem):
        return (
            pltpu.make_async_copy(
                w_hbm.at[idx_all.at[pl.ds(g * GT, 80)]],
                rows_v.at[pl.ds(0, 80)], sem),
            pltpu.make_async_copy(
                w_hbm.at[idx_all.at[pl.ds(g * GT + 80, 80)]],
                rows_v.at[pl.ds(80, 80)], sem),
        )

    def start_gather(g, rows_v, sem):
        for d in gather_descs(g, rows_v, sem):
            d.start()

    def wait_gather(g, rows_v, sem):
        for d in gather_descs(g, rows_v, sem):
            d.wait()

    def out_desc(g, out_v, osem):
        return pltpu.make_async_copy(
            out_v, out_hbm.at[pl.ds((seq_base + g * G) * EMB_D, G * EMB_D)],
            osem)

    pltpu.sync_copy(x_hbm.at[pl.ds(seq_base * SEQ_L, SEQ_PER_W * SEQ_L)],
                    idx_all)
    start_gather(0, rows0, sem0)
    start_gather(1, rows1, sem1)

    def seq_body(s, carry, g, rows_v, out_v):
        t0 = g * GT + s * SEQ_L
        c1 = idx_all[pl.ds(t0, 16)]
        c2 = idx_all[pl.ds(t0 + 4, 16)]
        one = jnp.full((16,), 1.0, jnp.float32)
        zero = jnp.full((16,), 0.0, jnp.float32)
        m1 = jnp.where(c1 != 0, one, zero)
        m2 = jnp.where((c2 != 0) & (lane >= 12), one, zero)
        cnt = m1 + m2
        for sh in (1, 2, 4, 8):
            cnt = cnt + cnt.at[lane ^ sh].get(mode="promise_in_bounds")
        inv = one / jnp.maximum(cnt, one)
        r0 = s * SEQ_L
        o0 = s * EMB_D
        for j in range(N_CHUNK):
            off = _CHUNK_OFF[j]
            vals = [rows_v[r0 + l, pl.ds(off, 16)] for l in range(SEQ_L)]
            while len(vals) > 1:
                vals = [vals[k] + vals[k + 1] for k in range(0, len(vals) - 1, 2)] + (
                    [vals[-1]] if len(vals) % 2 else [])
            out_v[pl.ds(o0 + off, 16)] = vals[0] * inv
        return carry

    def pair_body(i, carry):
        for p in (0, 1):
            g = 2 * i + p
            rows_v = rows0 if p == 0 else rows1
            sem = sem0 if p == 0 else sem1
            out_v = out0 if p == 0 else out1
            osem = osem0 if p == 0 else osem1
            wait_gather(g, rows_v, sem)

            @pl.when(g >= 2)
            def _():
                out_desc(g - 2, out_v, osem).wait()

            lax.fori_loop(0, G,
                          functools.partial(seq_body, g=g, rows_v=rows_v,
                                            out_v=out_v), 0)
            out_desc(g, out_v, osem).start()
            nxt = g + 2

            @pl.when(nxt < NG)
            def _():
                start_gather(nxt, rows_v, sem)
        return carry

    lax.fori_loop(0, NG // 2, pair_body, 0)
    out_desc(NG - 2, out0, osem0).wait()
    out_desc(NG - 1, out1, osem1).wait()


_emb = functools.partial(
    pl.kernel,
    out_type=jax.ShapeDtypeStruct((N_SEQ * EMB_D,), jnp.float32),
    mesh=plsc.VectorSubcoreMesh(core_axis_name="c", subcore_axis_name="s"),
    scratch_types=[
        pltpu.VMEM((SEQ_PER_W * SEQ_L,), jnp.int32),
        pltpu.VMEM((GT, EMB_D_PAD), jnp.float32),
        pltpu.VMEM((GT, EMB_D_PAD), jnp.float32),
        pltpu.VMEM((G * EMB_D,), jnp.float32),
        pltpu.VMEM((G * EMB_D,), jnp.float32),
        pltpu.SemaphoreType.DMA,
        pltpu.SemaphoreType.DMA,
        pltpu.SemaphoreType.DMA,
        pltpu.SemaphoreType.DMA,
    ],
    compiler_params=pltpu.CompilerParams(use_tc_tiling_on_sc=False),
)(_emb_body)


def kernel(x, W):
    b, nk, _ = x.shape
    w_pad = jnp.pad(W, ((0, 0), (0, EMB_D_PAD - EMB_D)))
    pooled = _emb(x.reshape(-1), w_pad)
    return pooled.reshape(b, nk, EMB_D)
